# baseline (speedup 1.0000x reference)
import functools

import jax
import jax.numpy as jnp
from jax import lax
from jax.experimental import pallas as pl
from jax.experimental.pallas import tpu as pltpu
from jax.experimental.pallas import tpu_sc as plsc

NC = 2
NS = 16
L = 16
NW = NC * NS


@functools.lru_cache(maxsize=None)
def _build(B, D):
    assert B % NW == 0
    bpw = B // NW
    chunk = min(128, bpw)
    nchunk = bpw // chunk
    assert bpw % chunk == 0 and D % L == 0

    mesh = plsc.VectorSubcoreMesh(
        core_axis_name="c", subcore_axis_name="s",
        num_cores=NC, num_subcores=NS,
    )

    @functools.partial(
        pl.kernel,
        out_type=jax.ShapeDtypeStruct((B,), jnp.float32),
        mesh=mesh,
        compiler_params=pltpu.CompilerParams(needs_layout_passes=False),
        scratch_types=[
            pltpu.VMEM((chunk,), jnp.int32),
            pltpu.VMEM((chunk,), jnp.int32),
            pltpu.VMEM((chunk, D), jnp.float32),
            pltpu.VMEM((chunk, D), jnp.float32),
            pltpu.VMEM((chunk,), jnp.float32),
            pltpu.VMEM((chunk,), jnp.float32),
            pltpu.VMEM((bpw,), jnp.float32),
            pltpu.VMEM((L,), jnp.float32),
            pltpu.SemaphoreType.DMA,
        ],
    )
    def mf_kernel(uidx_hbm, iidx_hbm, uemb_hbm, iemb_hbm, ubias_hbm,
                  ibias_hbm, mu_hbm, out_hbm,
                  uidx_v, iidx_v, urows_v, irows_v, ub_v, ib_v, out_v,
                  mu_v, sem):
        wid = lax.axis_index("s") * NC + lax.axis_index("c")
        base = wid * bpw
        pltpu.sync_copy(mu_hbm, mu_v)
        mu = mu_v[...]

        for k in range(nchunk):
            off = base + k * chunk
            pltpu.sync_copy(uidx_hbm.at[pl.ds(off, chunk)], uidx_v)
            pltpu.sync_copy(iidx_hbm.at[pl.ds(off, chunk)], iidx_v)
            cps = [
                pltpu.async_copy(uemb_hbm.at[uidx_v], urows_v, sem),
                pltpu.async_copy(iemb_hbm.at[iidx_v], irows_v, sem),
                pltpu.async_copy(ubias_hbm.at[uidx_v], ub_v, sem),
                pltpu.async_copy(ibias_hbm.at[iidx_v], ib_v, sem),
            ]
            for cp in cps:
                cp.wait()

            lanes = lax.iota(jnp.int32, L)
            for g in range(chunk // L):
                rows = g * L + lanes

                def d_body(d, acc):
                    dv = jnp.full((L,), d, jnp.int32)
                    u = plsc.load_gather(urows_v, [rows, dv])
                    iv = plsc.load_gather(irows_v, [rows, dv])
                    return acc + u * iv

                acc = lax.fori_loop(0, D, d_body,
                                    jnp.zeros((L,), jnp.float32), unroll=8)
                out_v[pl.ds(k * chunk + g * L, L)] = (
                    acc + ub_v[pl.ds(g * L, L)] + ib_v[pl.ds(g * L, L)] + mu)

        pltpu.sync_copy(out_v, out_hbm.at[pl.ds(base, bpw)])

    return mf_kernel


def kernel(user_indices, item_indices, user_emb, item_emb, user_bias,
           item_bias, mu):
    B = user_indices.shape[0]
    D = user_emb.shape[1]
    fn = _build(B, D)
    return fn(user_indices.astype(jnp.int32), item_indices.astype(jnp.int32),
              user_emb, item_emb, user_bias.reshape(-1),
              item_bias.reshape(-1), jnp.broadcast_to(mu, (L,)))


# speedup vs baseline: 1.0573x; 1.0573x over previous
import functools

import jax
import jax.numpy as jnp
from jax import lax
from jax.experimental import pallas as pl
from jax.experimental.pallas import tpu as pltpu
from jax.experimental.pallas import tpu_sc as plsc

NC = 2
NS = 16
L = 16
NW = NC * NS


@functools.lru_cache(maxsize=None)
def _build(B, D):
    assert B % NW == 0
    bpw = B // NW
    chunk = min(128, bpw)
    nchunk = bpw // chunk
    assert bpw % chunk == 0 and D % L == 0

    mesh = plsc.VectorSubcoreMesh(
        core_axis_name="c", subcore_axis_name="s",
        num_cores=NC, num_subcores=NS,
    )

    @functools.partial(
        pl.kernel,
        out_type=jax.ShapeDtypeStruct((B,), jnp.float32),
        mesh=mesh,
        compiler_params=pltpu.CompilerParams(needs_layout_passes=False),
        scratch_types=[
            pltpu.VMEM((bpw,), jnp.int32),
            pltpu.VMEM((bpw,), jnp.int32),
            pltpu.VMEM((chunk, D), jnp.float32),
            pltpu.VMEM((chunk, D), jnp.float32),
            pltpu.VMEM((chunk, D), jnp.float32),
            pltpu.VMEM((chunk, D), jnp.float32),
            pltpu.VMEM((chunk,), jnp.float32),
            pltpu.VMEM((chunk,), jnp.float32),
            pltpu.VMEM((chunk,), jnp.float32),
            pltpu.VMEM((chunk,), jnp.float32),
            pltpu.VMEM((bpw,), jnp.float32),
            pltpu.VMEM((L,), jnp.float32),
            pltpu.SemaphoreType.DMA,
            pltpu.SemaphoreType.DMA,
        ],
    )
    def mf_kernel(uidx_hbm, iidx_hbm, uemb_hbm, iemb_hbm, ubias_hbm,
                  ibias_hbm, mu_hbm, out_hbm,
                  uidx_v, iidx_v, ur0, ur1, ir0, ir1, ub0, ub1, ib0, ib1,
                  out_v, mu_v, sem0, sem1):
        wid = lax.axis_index("s") * NC + lax.axis_index("c")
        base = wid * bpw
        urows = (ur0, ur1)
        irows = (ir0, ir1)
        ubs = (ub0, ub1)
        ibs = (ib0, ib1)
        sems = (sem0, sem1)

        pltpu.sync_copy(mu_hbm, mu_v)
        mu = mu_v[...]
        pltpu.sync_copy(uidx_hbm.at[pl.ds(base, bpw)], uidx_v)
        pltpu.sync_copy(iidx_hbm.at[pl.ds(base, bpw)], iidx_v)

        def fire(k):
            p = k % 2
            sl = pl.ds(k * chunk, chunk)
            s = sems[p]
            return [
                pltpu.async_copy(uemb_hbm.at[uidx_v.at[sl]], urows[p], s),
                pltpu.async_copy(iemb_hbm.at[iidx_v.at[sl]], irows[p], s),
                pltpu.async_copy(ubias_hbm.at[uidx_v.at[sl]], ubs[p], s),
                pltpu.async_copy(ibias_hbm.at[iidx_v.at[sl]], ibs[p], s),
            ]

        cps = fire(0)
        lanes = lax.iota(jnp.int32, L)
        for k in range(nchunk):
            nxt = fire(k + 1) if k + 1 < nchunk else None
            for cp in cps:
                cp.wait()
            p = k % 2
            for g in range(chunk // L):
                rows = g * L + lanes

                def d_body(d, acc):
                    dv = jnp.full((L,), d, jnp.int32)
                    u = plsc.load_gather(urows[p], [rows, dv])
                    iv = plsc.load_gather(irows[p], [rows, dv])
                    return acc + u * iv

                acc = lax.fori_loop(0, D, d_body,
                                    jnp.zeros((L,), jnp.float32), unroll=8)
                out_v[pl.ds(k * chunk + g * L, L)] = (
                    acc + ubs[p][pl.ds(g * L, L)] + ibs[p][pl.ds(g * L, L)]
                    + mu)
            cps = nxt

        pltpu.sync_copy(out_v, out_hbm.at[pl.ds(base, bpw)])

    return mf_kernel


def kernel(user_indices, item_indices, user_emb, item_emb, user_bias,
           item_bias, mu):
    B = user_indices.shape[0]
    D = user_emb.shape[1]
    fn = _build(B, D)
    return fn(user_indices.astype(jnp.int32), item_indices.astype(jnp.int32),
              user_emb, item_emb, user_bias.reshape(-1),
              item_bias.reshape(-1), jnp.broadcast_to(mu, (L,)))


# speedup vs baseline: 1.1441x; 1.0821x over previous
import functools

import jax
import jax.numpy as jnp
from jax import lax
from jax.experimental import pallas as pl
from jax.experimental.pallas import tpu as pltpu
from jax.experimental.pallas import tpu_sc as plsc

NC = 2
NS = 16
L = 16
NW = NC * NS


@functools.lru_cache(maxsize=None)
def _build(B, D):
    assert B % NW == 0
    bpw = B // NW
    chunk = min(128, bpw)
    nchunk = bpw // chunk
    assert bpw % chunk == 0 and D % L == 0 and D % 8 == 0

    mesh = plsc.VectorSubcoreMesh(
        core_axis_name="c", subcore_axis_name="s",
        num_cores=NC, num_subcores=NS,
    )

    @functools.partial(
        pl.kernel,
        out_type=jax.ShapeDtypeStruct((B,), jnp.float32),
        mesh=mesh,
        compiler_params=pltpu.CompilerParams(needs_layout_passes=False),
        scratch_types=[
            pltpu.VMEM((bpw,), jnp.int32),
            pltpu.VMEM((bpw,), jnp.int32),
            pltpu.VMEM((chunk, D), jnp.float32),
            pltpu.VMEM((chunk, D), jnp.float32),
            pltpu.VMEM((chunk, D), jnp.float32),
            pltpu.VMEM((chunk, D), jnp.float32),
            pltpu.VMEM((chunk,), jnp.float32),
            pltpu.VMEM((chunk,), jnp.float32),
            pltpu.VMEM((chunk,), jnp.float32),
            pltpu.VMEM((chunk,), jnp.float32),
            pltpu.VMEM((bpw,), jnp.float32),
            pltpu.VMEM((L,), jnp.float32),
            pltpu.SemaphoreType.DMA,
            pltpu.SemaphoreType.DMA,
        ],
    )
    def mf_kernel(uidx_hbm, iidx_hbm, uemb_hbm, iemb_hbm, ubias_hbm,
                  ibias_hbm, mu_hbm, out_hbm,
                  uidx_v, iidx_v, ur0, ur1, ir0, ir1, ub0, ub1, ib0, ib1,
                  out_v, mu_v, sem0, sem1):
        wid = lax.axis_index("s") * NC + lax.axis_index("c")
        base = wid * bpw
        urows = (ur0, ur1)
        irows = (ir0, ir1)
        ubs = (ub0, ub1)
        ibs = (ib0, ib1)
        sems = (sem0, sem1)

        pltpu.sync_copy(mu_hbm, mu_v)
        mu = mu_v[...]
        pltpu.sync_copy(uidx_hbm.at[pl.ds(base, bpw)], uidx_v)
        pltpu.sync_copy(iidx_hbm.at[pl.ds(base, bpw)], iidx_v)

        def fire(k):
            p = k % 2
            sl = pl.ds(k * chunk, chunk)
            s = sems[p]
            return [
                pltpu.async_copy(uemb_hbm.at[uidx_v.at[sl]], urows[p], s),
                pltpu.async_copy(iemb_hbm.at[iidx_v.at[sl]], irows[p], s),
                pltpu.async_copy(ubias_hbm.at[uidx_v.at[sl]], ubs[p], s),
                pltpu.async_copy(ibias_hbm.at[iidx_v.at[sl]], ibs[p], s),
            ]

        cps = fire(0)
        lanes = lax.iota(jnp.int32, L)
        zero = jnp.zeros((L,), jnp.float32)
        for k in range(nchunk):
            nxt = fire(k + 1) if k + 1 < nchunk else None
            for cp in cps:
                cp.wait()
            p = k % 2
            ur, ir, ub, ib = urows[p], irows[p], ubs[p], ibs[p]

            def group_body(g, _):
                goff = pl.multiple_of(g * L, L)
                rows = goff + lanes

                def d_body(j, accs):
                    a0, a1, a2, a3 = accs
                    d0 = j * 8
                    ps = []
                    for q in range(8):
                        dv = jnp.full((L,), d0 + q, jnp.int32)
                        u = plsc.load_gather(ur, [rows, dv])
                        iv = plsc.load_gather(ir, [rows, dv])
                        ps.append(u * iv)
                    return (a0 + (ps[0] + ps[1]), a1 + (ps[2] + ps[3]),
                            a2 + (ps[4] + ps[5]), a3 + (ps[6] + ps[7]))

                a0, a1, a2, a3 = lax.fori_loop(0, D // 8, d_body,
                                               (zero, zero, zero, zero))
                acc = (a0 + a1) + (a2 + a3)
                out_v[pl.ds(k * chunk + goff, L)] = (
                    acc + ub[pl.ds(goff, L)] + ib[pl.ds(goff, L)] + mu)
                return 0

            lax.fori_loop(0, chunk // L, group_body, 0)
            cps = nxt

        pltpu.sync_copy(out_v, out_hbm.at[pl.ds(base, bpw)])

    return mf_kernel


def kernel(user_indices, item_indices, user_emb, item_emb, user_bias,
           item_bias, mu):
    B = user_indices.shape[0]
    D = user_emb.shape[1]
    fn = _build(B, D)
    return fn(user_indices.astype(jnp.int32), item_indices.astype(jnp.int32),
              user_emb, item_emb, user_bias.reshape(-1),
              item_bias.reshape(-1), jnp.broadcast_to(mu, (L,)))
